# Initial kernel scaffold; baseline (speedup 1.0000x reference)
import jax
import jax.numpy as jnp
from jax.experimental import pallas as pl


def kernel(x, gate_w, w1, w2):
    raise NotImplementedError("write your pallas kernel here")


# speedup vs baseline: 13.8595x; 13.8595x over previous
import functools

import jax
import jax.numpy as jnp
from jax import lax
from jax.experimental import pallas as pl
from jax.experimental.pallas import tpu as pltpu
from jax.experimental.pallas import tpu_sc as plsc

_E = 64
_C = 768
_H = 3072
_S = 2048
_CH = 64
_LOG2_CH = 6
_Q = 8
_NPAD = 2624
_NW = 32
_BPW = _S // _NW


def _gate_body(x_ref, gw_ref, logits_ref, dest_ref, val_ref, cnt_ref, off_ref):
    xf = x_ref[...]
    logits = lax.dot_general(
        xf, gw_ref[...], (((1,), (1,)), ((), ())),
        preferred_element_type=jnp.float32)
    logits_ref[...] = logits

    m = jnp.max(logits, axis=1, keepdims=True)
    ssum = jnp.sum(jnp.exp(logits - m), axis=1, keepdims=True)
    top_val = 1.0 / ssum

    lane = lax.broadcasted_iota(jnp.int32, (_S, _E), 1)
    ids = jnp.min(jnp.where(logits == m, lane, _E), axis=1, keepdims=True)
    onehot = (lane == ids).astype(jnp.float32)

    counts = jnp.sum(onehot, axis=0, keepdims=True)

    r = lax.broadcasted_iota(jnp.int32, (128, 128), 0)
    c = lax.broadcasted_iota(jnp.int32, (128, 128), 1)
    tri = (c < r).astype(jnp.float32)
    blocks = []
    carry = jnp.zeros((1, _E), jnp.float32)
    for b in range(_S // 128):
        blk = onehot[b * 128:(b + 1) * 128]
        rb = lax.dot_general(tri, blk, (((1,), (0,)), ((), ())),
                             preferred_element_type=jnp.float32) + carry
        blocks.append(rb)
        carry = carry + jnp.sum(blk, axis=0, keepdims=True)
    rank_m = jnp.concatenate(blocks, axis=0)
    rank = jnp.sum(rank_m * onehot, axis=1, keepdims=True)

    ci = counts.astype(jnp.int32)
    pci = ((ci + (_Q - 1)) >> 3) << 3
    re = lax.broadcasted_iota(jnp.int32, (_E, _E), 0)
    ce = lax.broadcasted_iota(jnp.int32, (_E, _E), 1)
    upper = (re < ce).astype(jnp.float32)
    offs = lax.dot_general(pci.astype(jnp.float32), upper,
                           (((1,), (0,)), ((), ())),
                           preferred_element_type=jnp.float32)

    off_tok = jnp.sum(onehot * offs, axis=1, keepdims=True)
    dest_ref[...] = (off_tok + rank).astype(jnp.int32)
    val_ref[...] = jnp.broadcast_to(top_val, (_S, 128))
    cnt_ref[...] = ci
    off_ref[...] = offs.astype(jnp.int32)


def _erf(z):
    az = jnp.abs(z)
    t = 1.0 / (1.0 + 0.3275911 * az)
    poly = t * (0.254829592 + t * (-0.284496736 + t * (1.421413741
               + t * (-1.453152027 + t * 1.061405429))))
    e = 1.0 - poly * jnp.exp(-az * az)
    return jnp.where(z < 0, -e, e)


def _gelu(z):
    return 0.5 * z * (1.0 + _erf(z * 0.7071067811865476))


def _gemm_body(cnt_ref, off_ref, xs_ref, dws_ref, w1_ref, w2_ref, ys_ref):
    e = pl.program_id(0)
    start = off_ref[0, e]
    cnt = cnt_ref[0, e]
    nch = (cnt + (_CH - 1)) >> _LOG2_CH

    def chunk(i, _):
        off = pl.multiple_of(start + i * _CH, _Q)
        xa = xs_ref[pl.ds(off, _CH), :]
        h = _gelu(jnp.dot(xa, w1_ref[...],
                          preferred_element_type=jnp.float32))
        o = jnp.dot(h, w2_ref[...], preferred_element_type=jnp.float32)
        dw = dws_ref[pl.ds(off, _CH), :][:, 0:1]
        ys_ref[pl.ds(off, _CH), :] = o * dw
        return 0

    lax.fori_loop(0, nch, chunk, 0)


@functools.cache
def _sc_kernels():
    mesh = plsc.VectorSubcoreMesh(core_axis_name="c", subcore_axis_name="s",
                                  num_cores=2, num_subcores=16)

    @functools.partial(
        pl.kernel,
        out_type=(jax.ShapeDtypeStruct((_NPAD, _C), jnp.float32),
                  jax.ShapeDtypeStruct((_NPAD, 128), jnp.float32)),
        mesh=mesh,
        scratch_types=[pltpu.VMEM((_BPW,), jnp.int32),
                       pltpu.VMEM((_BPW, _C), jnp.float32),
                       pltpu.VMEM((_BPW, 128), jnp.float32),
                       pltpu.SemaphoreType.DMA],
    )
    def sc_scatter(x_hbm, v_hbm, d_hbm, xs_hbm, dws_hbm, idx_v, rows_v,
                   vrows_v, sem):
        wid = lax.axis_index("s") * 2 + lax.axis_index("c")
        base = wid * _BPW
        pltpu.sync_copy(d_hbm.at[pl.ds(base, _BPW)], idx_v)
        pltpu.sync_copy(x_hbm.at[pl.ds(base, _BPW)], rows_v)
        pltpu.async_copy(rows_v, xs_hbm.at[idx_v], sem).wait()
        pltpu.sync_copy(v_hbm.at[pl.ds(base, _BPW)], vrows_v)
        pltpu.async_copy(vrows_v, dws_hbm.at[idx_v], sem).wait()

    @functools.partial(
        pl.kernel,
        out_type=jax.ShapeDtypeStruct((_S, _C), jnp.float32),
        mesh=mesh,
        scratch_types=[pltpu.VMEM((_BPW,), jnp.int32),
                       pltpu.VMEM((_BPW, _C), jnp.float32),
                       pltpu.SemaphoreType.DMA],
    )
    def sc_gather(ys_hbm, d_hbm, out_hbm, idx_v, rows_v, sem):
        wid = lax.axis_index("s") * 2 + lax.axis_index("c")
        base = wid * _BPW
        pltpu.sync_copy(d_hbm.at[pl.ds(base, _BPW)], idx_v)
        pltpu.async_copy(ys_hbm.at[idx_v], rows_v, sem).wait()
        pltpu.sync_copy(rows_v, out_hbm.at[pl.ds(base, _BPW)])

    return sc_scatter, sc_gather


def kernel(x, gate_w, w1, w2):
    B, T, C = x.shape
    flat = x.reshape(_S, _C)

    logits, dest, val16, counts, offs = pl.pallas_call(
        _gate_body,
        out_shape=(
            jax.ShapeDtypeStruct((_S, _E), jnp.float32),
            jax.ShapeDtypeStruct((_S, 1), jnp.int32),
            jax.ShapeDtypeStruct((_S, 128), jnp.float32),
            jax.ShapeDtypeStruct((1, _E), jnp.int32),
            jax.ShapeDtypeStruct((1, _E), jnp.int32),
        ),
    )(flat, gate_w)

    sc_scatter, sc_gather = _sc_kernels()
    dest1 = dest.reshape(_S)
    xs, dws = sc_scatter(flat, val16, dest1)

    ys = pl.pallas_call(
        _gemm_body,
        grid=(_E,),
        in_specs=[
            pl.BlockSpec(memory_space=pltpu.SMEM),
            pl.BlockSpec(memory_space=pltpu.SMEM),
            pl.BlockSpec((_NPAD, _C), lambda e: (0, 0)),
            pl.BlockSpec((_NPAD, 128), lambda e: (0, 0)),
            pl.BlockSpec((None, _C, _H), lambda e: (e, 0, 0)),
            pl.BlockSpec((None, _H, _C), lambda e: (e, 0, 0)),
        ],
        out_specs=pl.BlockSpec((_NPAD, _C), lambda e: (0, 0)),
        out_shape=jax.ShapeDtypeStruct((_NPAD, _C), jnp.float32),
        compiler_params=pltpu.CompilerParams(
            dimension_semantics=("arbitrary",)),
    )(counts, offs, xs, dws, w1, w2)

    out = sc_gather(ys, dest1)
    return out.reshape(B, T, C), logits


# speedup vs baseline: 14.2479x; 1.0280x over previous
import functools

import jax
import jax.numpy as jnp
from jax import lax
from jax.experimental import pallas as pl
from jax.experimental.pallas import tpu as pltpu
from jax.experimental.pallas import tpu_sc as plsc

_E = 64
_C = 768
_H = 3072
_S = 2048
_CH = 64
_LOG2_CH = 6
_Q = 8
_NPAD = 2624
_NW = 32
_BPW = _S // _NW


def _gate_body(x_ref, gw_ref, logits_ref, dest_ref, val_ref, cnt_ref, off_ref):
    xf = x_ref[...]
    logits = lax.dot_general(
        xf, gw_ref[...], (((1,), (1,)), ((), ())),
        preferred_element_type=jnp.float32)
    logits_ref[...] = logits

    m = jnp.max(logits, axis=1, keepdims=True)
    ssum = jnp.sum(jnp.exp(logits - m), axis=1, keepdims=True)
    top_val = 1.0 / ssum

    lane = lax.broadcasted_iota(jnp.int32, (_S, _E), 1)
    ids = jnp.min(jnp.where(logits == m, lane, _E), axis=1, keepdims=True)
    onehot = (lane == ids).astype(jnp.float32)

    counts = jnp.sum(onehot, axis=0, keepdims=True)

    r = lax.broadcasted_iota(jnp.int32, (128, 128), 0)
    c = lax.broadcasted_iota(jnp.int32, (128, 128), 1)
    tri = (c < r).astype(jnp.float32)
    blocks = []
    carry = jnp.zeros((1, _E), jnp.float32)
    for b in range(_S // 128):
        blk = onehot[b * 128:(b + 1) * 128]
        rb = lax.dot_general(tri, blk, (((1,), (0,)), ((), ())),
                             preferred_element_type=jnp.float32) + carry
        blocks.append(rb)
        carry = carry + jnp.sum(blk, axis=0, keepdims=True)
    rank_m = jnp.concatenate(blocks, axis=0)
    rank = jnp.sum(rank_m * onehot, axis=1, keepdims=True)

    ci = counts.astype(jnp.int32)
    pci = ((ci + (_Q - 1)) >> 3) << 3
    re = lax.broadcasted_iota(jnp.int32, (_E, _E), 0)
    ce = lax.broadcasted_iota(jnp.int32, (_E, _E), 1)
    upper = (re < ce).astype(jnp.float32)
    offs = lax.dot_general(pci.astype(jnp.float32), upper,
                           (((1,), (0,)), ((), ())),
                           preferred_element_type=jnp.float32)

    off_tok = jnp.sum(onehot * offs, axis=1, keepdims=True)
    dest_ref[...] = (off_tok + rank).astype(jnp.int32)
    val_ref[...] = jnp.broadcast_to(top_val, (_S, 128))
    cnt_ref[...] = ci
    off_ref[...] = offs.astype(jnp.int32)


def _erf(z):
    az = jnp.abs(z)
    t = 1.0 / (1.0 + 0.3275911 * az)
    poly = t * (0.254829592 + t * (-0.284496736 + t * (1.421413741
               + t * (-1.453152027 + t * 1.061405429))))
    e = 1.0 - poly * jnp.exp(-az * az)
    return jnp.where(z < 0, -e, e)


def _gelu(z):
    return 0.5 * z * (1.0 + _erf(z * 0.7071067811865476))


def _gemm_body(cnt_ref, off_ref, w1_ref, w2_ref, xs_ref, dws_ref, ys_ref):
    e = pl.program_id(0)
    j = pl.program_id(1)
    start = off_ref[0, e]
    cnt = cnt_ref[0, e]
    nch = (cnt + (_CH - 1)) >> _LOG2_CH

    def chunk(i, _):
        off = pl.multiple_of(start + i * _CH, _Q)
        xa = xs_ref[pl.ds(off, _CH), :]
        h = _gelu(jnp.dot(xa, w1_ref[...],
                          preferred_element_type=jnp.float32))
        o = jnp.dot(h, w2_ref[...], preferred_element_type=jnp.float32)
        dw = dws_ref[pl.ds(off, _CH), :][:, 0:1]
        o = o * dw

        @pl.when(j == 0)
        def _():
            ys_ref[pl.ds(off, _CH), :] = o

        @pl.when(j == 1)
        def _():
            ys_ref[pl.ds(off, _CH), :] += o

        return 0

    lax.fori_loop(0, nch, chunk, 0)


@functools.cache
def _sc_kernels():
    mesh = plsc.VectorSubcoreMesh(core_axis_name="c", subcore_axis_name="s",
                                  num_cores=2, num_subcores=16)

    @functools.partial(
        pl.kernel,
        out_type=(jax.ShapeDtypeStruct((_NPAD, _C), jnp.float32),
                  jax.ShapeDtypeStruct((_NPAD, 128), jnp.float32)),
        mesh=mesh,
        scratch_types=[pltpu.VMEM((_BPW,), jnp.int32),
                       pltpu.VMEM((_BPW, _C), jnp.float32),
                       pltpu.VMEM((_BPW, 128), jnp.float32),
                       pltpu.SemaphoreType.DMA],
    )
    def sc_scatter(x_hbm, v_hbm, d_hbm, xs_hbm, dws_hbm, idx_v, rows_v,
                   vrows_v, sem):
        wid = lax.axis_index("s") * 2 + lax.axis_index("c")
        base = wid * _BPW
        pltpu.sync_copy(d_hbm.at[pl.ds(base, _BPW)], idx_v)
        pltpu.sync_copy(x_hbm.at[pl.ds(base, _BPW)], rows_v)
        pltpu.async_copy(rows_v, xs_hbm.at[idx_v], sem).wait()
        pltpu.sync_copy(v_hbm.at[pl.ds(base, _BPW)], vrows_v)
        pltpu.async_copy(vrows_v, dws_hbm.at[idx_v], sem).wait()

    @functools.partial(
        pl.kernel,
        out_type=jax.ShapeDtypeStruct((_S, _C), jnp.float32),
        mesh=mesh,
        scratch_types=[pltpu.VMEM((_BPW,), jnp.int32),
                       pltpu.VMEM((_BPW, _C), jnp.float32),
                       pltpu.SemaphoreType.DMA],
    )
    def sc_gather(ys_hbm, d_hbm, out_hbm, idx_v, rows_v, sem):
        wid = lax.axis_index("s") * 2 + lax.axis_index("c")
        base = wid * _BPW
        pltpu.sync_copy(d_hbm.at[pl.ds(base, _BPW)], idx_v)
        pltpu.async_copy(ys_hbm.at[idx_v], rows_v, sem).wait()
        pltpu.sync_copy(rows_v, out_hbm.at[pl.ds(base, _BPW)])

    return sc_scatter, sc_gather


def kernel(x, gate_w, w1, w2):
    B, T, C = x.shape
    flat = x.reshape(_S, _C)

    logits, dest, val16, counts, offs = pl.pallas_call(
        _gate_body,
        out_shape=(
            jax.ShapeDtypeStruct((_S, _E), jnp.float32),
            jax.ShapeDtypeStruct((_S, 1), jnp.int32),
            jax.ShapeDtypeStruct((_S, 128), jnp.float32),
            jax.ShapeDtypeStruct((1, _E), jnp.int32),
            jax.ShapeDtypeStruct((1, _E), jnp.int32),
        ),
    )(flat, gate_w)

    sc_scatter, sc_gather = _sc_kernels()
    dest1 = dest.reshape(_S)
    xs, dws = sc_scatter(flat, val16, dest1)

    ys = pl.pallas_call(
        _gemm_body,
        grid=(_E, 2),
        in_specs=[
            pl.BlockSpec(memory_space=pltpu.SMEM),
            pl.BlockSpec(memory_space=pltpu.SMEM),
            pl.BlockSpec((None, _C, _H // 2), lambda e, j: (e, 0, j)),
            pl.BlockSpec((None, _H // 2, _C), lambda e, j: (e, j, 0)),
            pl.BlockSpec((_NPAD, _C), lambda e, j: (0, 0)),
            pl.BlockSpec((_NPAD, 128), lambda e, j: (0, 0)),
        ],
        out_specs=pl.BlockSpec((_NPAD, _C), lambda e, j: (0, 0)),
        out_shape=jax.ShapeDtypeStruct((_NPAD, _C), jnp.float32),
        compiler_params=pltpu.CompilerParams(
            dimension_semantics=("arbitrary", "arbitrary")),
    )(counts, offs, w1, w2, xs, dws)

    out = sc_gather(ys, dest1)
    return out.reshape(B, T, C), logits


# speedup vs baseline: 14.4077x; 1.0112x over previous
import functools

import jax
import jax.numpy as jnp
from jax import lax
from jax.experimental import pallas as pl
from jax.experimental.pallas import tpu as pltpu
from jax.experimental.pallas import tpu_sc as plsc

_E = 64
_C = 768
_H = 3072
_S = 2048
_CH = 64
_LOG2_CH = 6
_Q = 8
_NPAD = 2560
_NW = 32
_BPW = _S // _NW
_CW = _C + 128


def _gate_body(x_ref, gw_ref, logits_ref, dest_ref, val_ref, cnt_ref, off_ref):
    xf = x_ref[...]
    logits = lax.dot_general(
        xf, gw_ref[...], (((1,), (1,)), ((), ())),
        preferred_element_type=jnp.float32)
    logits_ref[...] = logits

    m = jnp.max(logits, axis=1, keepdims=True)
    ssum = jnp.sum(jnp.exp(logits - m), axis=1, keepdims=True)
    top_val = 1.0 / ssum

    lane = lax.broadcasted_iota(jnp.int32, (_S, _E), 1)
    ids = jnp.min(jnp.where(logits == m, lane, _E), axis=1, keepdims=True)
    onehot = (lane == ids).astype(jnp.float32)

    counts = jnp.sum(onehot, axis=0, keepdims=True)

    r = lax.broadcasted_iota(jnp.int32, (128, 128), 0)
    c = lax.broadcasted_iota(jnp.int32, (128, 128), 1)
    tri = (c < r).astype(jnp.float32)
    blocks = []
    carry = jnp.zeros((1, _E), jnp.float32)
    for b in range(_S // 128):
        blk = onehot[b * 128:(b + 1) * 128]
        rb = lax.dot_general(tri, blk, (((1,), (0,)), ((), ())),
                             preferred_element_type=jnp.float32) + carry
        blocks.append(rb)
        carry = carry + jnp.sum(blk, axis=0, keepdims=True)
    rank_m = jnp.concatenate(blocks, axis=0)
    rank = jnp.sum(rank_m * onehot, axis=1, keepdims=True)

    ci = counts.astype(jnp.int32)
    pci = ((ci + (_Q - 1)) >> 3) << 3
    re = lax.broadcasted_iota(jnp.int32, (_E, _E), 0)
    ce = lax.broadcasted_iota(jnp.int32, (_E, _E), 1)
    upper = (re < ce).astype(jnp.float32)
    offs = lax.dot_general(pci.astype(jnp.float32), upper,
                           (((1,), (0,)), ((), ())),
                           preferred_element_type=jnp.float32)

    off_tok = jnp.sum(onehot * offs, axis=1, keepdims=True)
    dest_ref[...] = (off_tok + rank).astype(jnp.int32)
    val_ref[...] = jnp.broadcast_to(top_val, (_S, 128))
    cnt_ref[...] = ci
    off_ref[...] = offs.astype(jnp.int32)


def _erf(z):
    az = jnp.abs(z)
    t = 1.0 / (1.0 + 0.3275911 * az)
    poly = t * (0.254829592 + t * (-0.284496736 + t * (1.421413741
               + t * (-1.453152027 + t * 1.061405429))))
    e = 1.0 - poly * jnp.exp(-az * az)
    return jnp.where(z < 0, -e, e)


def _gelu(z):
    return 0.5 * z * (1.0 + _erf(z * 0.7071067811865476))


def _gemm_body(cnt_ref, off_ref, w1_ref, w2_ref, xs_ref, ys_ref):
    e = pl.program_id(0)
    j = pl.program_id(1)
    start = off_ref[0, e]
    cnt = cnt_ref[0, e]
    nch = (cnt + (_CH - 1)) >> _LOG2_CH

    def chunk(i, _):
        off = pl.multiple_of(start + i * _CH, _Q)
        slab = xs_ref[pl.ds(off, _CH), :]
        xa = slab[:, :_C]
        dw = slab[:, _C:_C + 1]
        h = _gelu(jnp.dot(xa, w1_ref[...],
                          preferred_element_type=jnp.float32))
        o = jnp.dot(h, w2_ref[...], preferred_element_type=jnp.float32)
        o = o * dw

        @pl.when(j == 0)
        def _():
            ys_ref[pl.ds(off, _CH), :] = o

        @pl.when(j == 1)
        def _():
            ys_ref[pl.ds(off, _CH), :] += o

        return 0

    lax.fori_loop(0, nch, chunk, 0)


@functools.cache
def _sc_kernels():
    mesh = plsc.VectorSubcoreMesh(core_axis_name="c", subcore_axis_name="s",
                                  num_cores=2, num_subcores=16)

    @functools.partial(
        pl.kernel,
        out_type=jax.ShapeDtypeStruct((_NPAD, _CW), jnp.float32),
        mesh=mesh,
        scratch_types=[pltpu.VMEM((_BPW,), jnp.int32),
                       pltpu.VMEM((_BPW, _CW), jnp.float32),
                       pltpu.SemaphoreType.DMA,
                       pltpu.SemaphoreType.DMA],
    )
    def sc_scatter(x_hbm, v_hbm, d_hbm, xs_hbm, idx_v, rows_v, sem, sem2):
        wid = lax.axis_index("s") * 2 + lax.axis_index("c")
        base = wid * _BPW
        cx = pltpu.async_copy(x_hbm.at[pl.ds(base, _BPW)],
                              rows_v.at[:, pl.ds(0, _C)], sem)
        cv = pltpu.async_copy(v_hbm.at[pl.ds(base, _BPW)],
                              rows_v.at[:, pl.ds(_C, 128)], sem2)
        pltpu.sync_copy(d_hbm.at[pl.ds(base, _BPW)], idx_v)
        cx.wait()
        cv.wait()
        pltpu.async_copy(rows_v, xs_hbm.at[idx_v], sem).wait()

    @functools.partial(
        pl.kernel,
        out_type=jax.ShapeDtypeStruct((_S, _C), jnp.float32),
        mesh=mesh,
        scratch_types=[pltpu.VMEM((_BPW,), jnp.int32),
                       pltpu.VMEM((_BPW, _C), jnp.float32),
                       pltpu.SemaphoreType.DMA],
    )
    def sc_gather(ys_hbm, d_hbm, out_hbm, idx_v, rows_v, sem):
        wid = lax.axis_index("s") * 2 + lax.axis_index("c")
        base = wid * _BPW
        pltpu.sync_copy(d_hbm.at[pl.ds(base, _BPW)], idx_v)
        pltpu.async_copy(ys_hbm.at[idx_v], rows_v, sem).wait()
        pltpu.sync_copy(rows_v, out_hbm.at[pl.ds(base, _BPW)])

    return sc_scatter, sc_gather


def kernel(x, gate_w, w1, w2):
    B, T, C = x.shape
    flat = x.reshape(_S, _C)

    logits, dest, val16, counts, offs = pl.pallas_call(
        _gate_body,
        out_shape=(
            jax.ShapeDtypeStruct((_S, _E), jnp.float32),
            jax.ShapeDtypeStruct((_S, 1), jnp.int32),
            jax.ShapeDtypeStruct((_S, 128), jnp.float32),
            jax.ShapeDtypeStruct((1, _E), jnp.int32),
            jax.ShapeDtypeStruct((1, _E), jnp.int32),
        ),
    )(flat, gate_w)

    sc_scatter, sc_gather = _sc_kernels()
    dest1 = dest.reshape(_S)
    xs = sc_scatter(flat, val16, dest1)

    ys = pl.pallas_call(
        _gemm_body,
        grid=(_E, 2),
        in_specs=[
            pl.BlockSpec(memory_space=pltpu.SMEM),
            pl.BlockSpec(memory_space=pltpu.SMEM),
            pl.BlockSpec((None, _C, _H // 2), lambda e, j: (e, 0, j)),
            pl.BlockSpec((None, _H // 2, _C), lambda e, j: (e, j, 0)),
            pl.BlockSpec((_NPAD, _CW), lambda e, j: (0, 0)),
        ],
        out_specs=pl.BlockSpec((_NPAD, _C), lambda e, j: (0, 0)),
        out_shape=jax.ShapeDtypeStruct((_NPAD, _C), jnp.float32),
        compiler_params=pltpu.CompilerParams(
            dimension_semantics=("arbitrary", "arbitrary")),
    )(counts, offs, w1, w2, xs)

    out = sc_gather(ys, dest1)
    return out.reshape(B, T, C), logits


# speedup vs baseline: 14.4720x; 1.0045x over previous
import functools

import jax
import jax.numpy as jnp
from jax import lax
from jax.experimental import pallas as pl
from jax.experimental.pallas import tpu as pltpu
from jax.experimental.pallas import tpu_sc as plsc

_E = 64
_C = 768
_H = 3072
_S = 2048
_CH = 64
_LOG2_CH = 6
_Q = 8
_NPAD = 2560
_NW = 32
_BPW = _S // _NW
_CW = _C + 128


def _gate_body(x_ref, gw_ref, logits_ref, dest_ref, val_ref, cnt_ref, off_ref):
    xf = x_ref[...]
    logits = lax.dot_general(
        xf, gw_ref[...], (((1,), (1,)), ((), ())),
        preferred_element_type=jnp.float32)
    logits_ref[...] = logits

    m = jnp.max(logits, axis=1, keepdims=True)
    ssum = jnp.sum(jnp.exp(logits - m), axis=1, keepdims=True)
    top_val = 1.0 / ssum

    lane = lax.broadcasted_iota(jnp.int32, (_S, _E), 1)
    ids = jnp.min(jnp.where(logits == m, lane, _E), axis=1, keepdims=True)
    onehot = (lane == ids).astype(jnp.float32)

    counts = jnp.sum(onehot, axis=0, keepdims=True)

    r = lax.broadcasted_iota(jnp.int32, (128, 128), 0)
    c = lax.broadcasted_iota(jnp.int32, (128, 128), 1)
    tri = (c < r).astype(jnp.float32)
    blocks = []
    carry = jnp.zeros((1, _E), jnp.float32)
    for b in range(_S // 128):
        blk = onehot[b * 128:(b + 1) * 128]
        rb = lax.dot_general(tri, blk, (((1,), (0,)), ((), ())),
                             preferred_element_type=jnp.float32) + carry
        blocks.append(rb)
        carry = carry + jnp.sum(blk, axis=0, keepdims=True)
    rank_m = jnp.concatenate(blocks, axis=0)
    rank = jnp.sum(rank_m * onehot, axis=1, keepdims=True)

    ci = counts.astype(jnp.int32)
    pci = ((ci + (_Q - 1)) >> 3) << 3
    re = lax.broadcasted_iota(jnp.int32, (_E, _E), 0)
    ce = lax.broadcasted_iota(jnp.int32, (_E, _E), 1)
    upper = (re < ce).astype(jnp.float32)
    offs = lax.dot_general(pci.astype(jnp.float32), upper,
                           (((1,), (0,)), ((), ())),
                           preferred_element_type=jnp.float32)

    off_tok = jnp.sum(onehot * offs, axis=1, keepdims=True)
    dest_ref[...] = (off_tok + rank).astype(jnp.int32)
    val_ref[...] = jnp.broadcast_to(top_val, (_S, 128))
    cnt_ref[...] = ci
    off_ref[...] = offs.astype(jnp.int32)


def _erf(z):
    az = jnp.abs(z)
    t = 1.0 / (1.0 + 0.3275911 * az)
    poly = t * (0.254829592 + t * (-0.284496736 + t * (1.421413741
               + t * (-1.453152027 + t * 1.061405429))))
    e = 1.0 - poly * jnp.exp(-az * az)
    return jnp.where(z < 0, -e, e)


def _gelu(z):
    return 0.5 * z * (1.0 + _erf(z * 0.7071067811865476))


def _gemm_body(cnt_ref, off_ref, w1_ref, w2_ref, xs_ref, ys_ref):
    e = pl.program_id(0)
    j = pl.program_id(1)
    start = off_ref[0, e]
    cnt = cnt_ref[0, e]
    nch = (cnt + (_CH - 1)) >> _LOG2_CH

    def chunk(i, _):
        off = pl.multiple_of(start + i * _CH, _Q)
        slab = xs_ref[pl.ds(off, _CH), :]
        xa = slab[:, :_C]
        dw = slab[:, _C:_C + 1]
        h = _gelu(jnp.dot(xa, w1_ref[...],
                          preferred_element_type=jnp.float32))
        o = jnp.dot(h, w2_ref[...], preferred_element_type=jnp.float32)
        o = o * dw

        @pl.when(j == 0)
        def _():
            ys_ref[pl.ds(off, _CH), :] = o

        @pl.when(j == 1)
        def _():
            ys_ref[pl.ds(off, _CH), :] += o

        return 0

    lax.fori_loop(0, nch, chunk, 0)


@functools.cache
def _sc_kernels():
    mesh = plsc.VectorSubcoreMesh(core_axis_name="c", subcore_axis_name="s",
                                  num_cores=2, num_subcores=16)

    hw = _BPW // 2

    @functools.partial(
        pl.kernel,
        out_type=jax.ShapeDtypeStruct((_NPAD, _CW), jnp.float32),
        mesh=mesh,
        scratch_types=[pltpu.VMEM((hw,), jnp.int32),
                       pltpu.VMEM((hw,), jnp.int32),
                       pltpu.VMEM((hw, _CW), jnp.float32),
                       pltpu.VMEM((hw, _CW), jnp.float32),
                       pltpu.SemaphoreType.DMA,
                       pltpu.SemaphoreType.DMA,
                       pltpu.SemaphoreType.DMA,
                       pltpu.SemaphoreType.DMA],
    )
    def sc_scatter(x_hbm, v_hbm, d_hbm, xs_hbm, idx_a, idx_b, rows_a, rows_b,
                   sa, sb, sc, sd):
        wid = lax.axis_index("s") * 2 + lax.axis_index("c")
        base = wid * _BPW
        cxa = pltpu.async_copy(x_hbm.at[pl.ds(base, hw)],
                               rows_a.at[:, pl.ds(0, _C)], sa)
        cva = pltpu.async_copy(v_hbm.at[pl.ds(base, hw)],
                               rows_a.at[:, pl.ds(_C, 128)], sa)
        cxb = pltpu.async_copy(x_hbm.at[pl.ds(base + hw, hw)],
                               rows_b.at[:, pl.ds(0, _C)], sb)
        cvb = pltpu.async_copy(v_hbm.at[pl.ds(base + hw, hw)],
                               rows_b.at[:, pl.ds(_C, 128)], sb)
        pltpu.sync_copy(d_hbm.at[pl.ds(base, hw)], idx_a)
        pltpu.sync_copy(d_hbm.at[pl.ds(base + hw, hw)], idx_b)
        cxa.wait()
        cva.wait()
        wa = pltpu.async_copy(rows_a, xs_hbm.at[idx_a], sc)
        cxb.wait()
        cvb.wait()
        wb = pltpu.async_copy(rows_b, xs_hbm.at[idx_b], sd)
        wa.wait()
        wb.wait()

    @functools.partial(
        pl.kernel,
        out_type=jax.ShapeDtypeStruct((_S, _C), jnp.float32),
        mesh=mesh,
        scratch_types=[pltpu.VMEM((hw,), jnp.int32),
                       pltpu.VMEM((hw,), jnp.int32),
                       pltpu.VMEM((hw, _C), jnp.float32),
                       pltpu.VMEM((hw, _C), jnp.float32),
                       pltpu.SemaphoreType.DMA,
                       pltpu.SemaphoreType.DMA,
                       pltpu.SemaphoreType.DMA,
                       pltpu.SemaphoreType.DMA],
    )
    def sc_gather(ys_hbm, d_hbm, out_hbm, idx_a, idx_b, rows_a, rows_b,
                  sa, sb, sc, sd):
        wid = lax.axis_index("s") * 2 + lax.axis_index("c")
        base = wid * _BPW
        pltpu.sync_copy(d_hbm.at[pl.ds(base, hw)], idx_a)
        pltpu.sync_copy(d_hbm.at[pl.ds(base + hw, hw)], idx_b)
        ga = pltpu.async_copy(ys_hbm.at[idx_a], rows_a, sa)
        gb = pltpu.async_copy(ys_hbm.at[idx_b], rows_b, sb)
        ga.wait()
        wa = pltpu.async_copy(rows_a, out_hbm.at[pl.ds(base, hw)], sc)
        gb.wait()
        wb = pltpu.async_copy(rows_b, out_hbm.at[pl.ds(base + hw, hw)], sd)
        wa.wait()
        wb.wait()

    return sc_scatter, sc_gather


def kernel(x, gate_w, w1, w2):
    B, T, C = x.shape
    flat = x.reshape(_S, _C)

    logits, dest, val16, counts, offs = pl.pallas_call(
        _gate_body,
        out_shape=(
            jax.ShapeDtypeStruct((_S, _E), jnp.float32),
            jax.ShapeDtypeStruct((_S, 1), jnp.int32),
            jax.ShapeDtypeStruct((_S, 128), jnp.float32),
            jax.ShapeDtypeStruct((1, _E), jnp.int32),
            jax.ShapeDtypeStruct((1, _E), jnp.int32),
        ),
    )(flat, gate_w)

    sc_scatter, sc_gather = _sc_kernels()
    dest1 = dest.reshape(_S)
    xs = sc_scatter(flat, val16, dest1)

    ys = pl.pallas_call(
        _gemm_body,
        grid=(_E, 2),
        in_specs=[
            pl.BlockSpec(memory_space=pltpu.SMEM),
            pl.BlockSpec(memory_space=pltpu.SMEM),
            pl.BlockSpec((None, _C, _H // 2), lambda e, j: (e, 0, j)),
            pl.BlockSpec((None, _H // 2, _C), lambda e, j: (e, j, 0)),
            pl.BlockSpec((_NPAD, _CW), lambda e, j: (0, 0)),
        ],
        out_specs=pl.BlockSpec((_NPAD, _C), lambda e, j: (0, 0)),
        out_shape=jax.ShapeDtypeStruct((_NPAD, _C), jnp.float32),
        compiler_params=pltpu.CompilerParams(
            dimension_semantics=("arbitrary", "arbitrary")),
    )(counts, offs, w1, w2, xs)

    out = sc_gather(ys, dest1)
    return out.reshape(B, T, C), logits
